# baseline (speedup 1.0000x reference)
import functools

import jax
import jax.numpy as jnp
import numpy as np
from jax import lax
from jax.experimental import pallas as pl
from jax.experimental.pallas import tpu as pltpu
from jax.experimental.pallas import tpu_sc as plsc

B = 16384
DIM = 128

_PERM = np.concatenate(
    [32 * g + np.array([(j % 2) * 16 + j // 2 for j in range(32)])
     for g in range(DIM // 32)])


def _sc_info():
    try:
        info = plsc.get_sparse_core_info()
        return info.num_cores, info.num_subcores
    except Exception:
        return 2, 16


def _make_gather(nb):
    NC, NS = _sc_info()
    NW = NC * NS
    b_per_w = nb // NW
    C = min(64, b_per_w)
    n_chunks = b_per_w // C

    mesh = plsc.VectorSubcoreMesh(core_axis_name="c", subcore_axis_name="s")

    @functools.partial(
        pl.kernel,
        mesh=mesh,
        compiler_params=pltpu.CompilerParams(needs_layout_passes=False),
        out_type=[
            jax.ShapeDtypeStruct((nb, DIM // 2), jnp.int32),
            jax.ShapeDtypeStruct((nb, DIM // 2), jnp.int32),
        ],
        scratch_types=[
            pltpu.VMEM((n_chunks, C), jnp.int32),
            pltpu.VMEM((n_chunks, C), jnp.int32),
            pltpu.VMEM((n_chunks, C), jnp.int32),
            pltpu.VMEM((C, DIM), jnp.float32),
            pltpu.VMEM((C, DIM), jnp.float32),
            pltpu.VMEM((C, DIM), jnp.float32),
            pltpu.VMEM((C, DIM // 2), jnp.int32),
            pltpu.VMEM((C, DIM // 2), jnp.int32),
            pltpu.VMEM((C, DIM), jnp.float32),
            pltpu.VMEM((C, DIM), jnp.float32),
            pltpu.VMEM((C, DIM), jnp.float32),
            pltpu.VMEM((C, DIM // 2), jnp.int32),
            pltpu.VMEM((C, DIM // 2), jnp.int32),
            pltpu.SemaphoreType.DMA,
            pltpu.SemaphoreType.DMA,
            pltpu.SemaphoreType.DMA,
            pltpu.SemaphoreType.DMA,
            pltpu.SemaphoreType.DMA,
        ],
    )
    def gather_kernel(heads_hbm, rels_hbm, tails_hbm, ent_hbm, rel_hbm,
                      u_out, t_out, hidx, ridx, tidx,
                      hv0, rv0, tv0, ub0, tb0,
                      hv1, rv1, tv1, ub1, tb1,
                      isem, gsem0, gsem1, wsem0, wsem1):
        wid = lax.axis_index("s") * NC + lax.axis_index("c")
        base = wid * b_per_w
        slots = [(hv0, rv0, tv0, ub0, tb0, gsem0, wsem0),
                 (hv1, rv1, tv1, ub1, tb1, gsem1, wsem1)]

        idescs = []
        for ci in range(n_chunks):
            off = base + ci * C
            idescs.append(pltpu.async_copy(
                heads_hbm.at[pl.ds(off, C)], hidx.at[ci], isem))
            idescs.append(pltpu.async_copy(
                rels_hbm.at[pl.ds(off, C)], ridx.at[ci], isem))
            idescs.append(pltpu.async_copy(
                tails_hbm.at[pl.ds(off, C)], tidx.at[ci], isem))
        for d in idescs:
            d.wait()

        def fire(ci):
            hv, rv, tv, _, _, gsem, _ = slots[ci % 2]
            return (pltpu.async_copy(ent_hbm.at[hidx.at[ci]], hv, gsem),
                    pltpu.async_copy(rel_hbm.at[ridx.at[ci]], rv, gsem),
                    pltpu.async_copy(ent_hbm.at[tidx.at[ci]], tv, gsem))

        gd = {0: fire(0)}
        wd = {}
        for ci in range(n_chunks):
            hv, rv, tv, ub, tb, _, wsem = slots[ci % 2]
            off = base + ci * C
            if ci - 1 in wd:
                for d in wd.pop(ci - 1):
                    d.wait()
            if ci + 1 < n_chunks:
                gd[ci + 1] = fire(ci + 1)
            for d in gd.pop(ci):
                d.wait()

            def rne(x):
                v = plsc.bitcast(x, jnp.uint32)
                return (v + jnp.uint32(0x7FFF)
                        + ((v >> jnp.uint32(16)) & jnp.uint32(1))
                        ) >> jnp.uint32(16)

            def row_body(i, carry):
                for j in range(DIM // 32):
                    s0 = pl.ds(32 * j, 16)
                    s1 = pl.ds(32 * j + 16, 16)
                    sw = pl.ds(16 * j, 16)
                    ua = hv[i, s0] * rv[i, s0]
                    ubv = hv[i, s1] * rv[i, s1]
                    uw = rne(ua) | (rne(ubv) << jnp.uint32(16))
                    ub[i, sw] = plsc.bitcast(uw, jnp.int32)
                    tw = rne(tv[i, s0]) | (rne(tv[i, s1]) << jnp.uint32(16))
                    tb[i, sw] = plsc.bitcast(tw, jnp.int32)
                return carry
            lax.fori_loop(0, C, row_body, 0, unroll=2)

            wu = pltpu.async_copy(ub, u_out.at[pl.ds(off, C)], wsem)
            wt = pltpu.async_copy(tb, t_out.at[pl.ds(off, C)], wsem)
            wd[ci] = (wu, wt)
        for ci in list(wd):
            for d in wd.pop(ci):
                d.wait()

    return gather_kernel


def _score_body(u_ref, t_ref, w1_ref, w2_ref, b_ref, ones_ref, out_ref):
    u = u_ref[...]
    t = t_ref[...]
    acc = jnp.dot(u, w1_ref[...], preferred_element_type=jnp.float32)
    acc = acc + jnp.dot(t, w2_ref[...], preferred_element_type=jnp.float32)
    acc = acc + b_ref[...]
    m = acc * t.astype(jnp.float32)
    out_ref[...] = jnp.dot(m, ones_ref[...],
                           preferred_element_type=jnp.float32)


def _score_call(u, t, w1, w2, b, nb):
    RB = min(4096, nb)
    ones = jnp.ones((DIM, 8), jnp.float32)
    out2d = pl.pallas_call(
        _score_body,
        grid=(nb // RB,),
        in_specs=[
            pl.BlockSpec((RB, DIM), lambda i: (i, 0)),
            pl.BlockSpec((RB, DIM), lambda i: (i, 0)),
            pl.BlockSpec((DIM, DIM), lambda i: (0, 0)),
            pl.BlockSpec((DIM, DIM), lambda i: (0, 0)),
            pl.BlockSpec((DIM,), lambda i: (0,)),
            pl.BlockSpec((DIM, 8), lambda i: (0, 0)),
        ],
        out_specs=pl.BlockSpec((RB, 8), lambda i: (i, 0)),
        out_shape=jax.ShapeDtypeStruct((nb, 8), jnp.float32),
    )(u, t, w1, w2, b, ones)
    return out2d[:, 0]


NSLICES = 1


@jax.jit
def kernel(heads, rels, tails, entity_emb, rel_emb, W, b):
    nb = B // NSLICES
    gather = _make_gather(nb)
    perm = jnp.asarray(_PERM)
    w1 = W[:DIM][perm][:, perm].astype(jnp.bfloat16)
    w2 = W[DIM:][perm][:, perm].astype(jnp.bfloat16)
    bp = b[perm]
    outs = []
    for k in range(NSLICES):
        sl = slice(k * nb, (k + 1) * nb)
        u32, t32 = gather(heads[sl], rels[sl], tails[sl],
                          entity_emb, rel_emb)
        u = lax.bitcast_convert_type(u32, jnp.bfloat16).reshape(nb, DIM)
        t = lax.bitcast_convert_type(t32, jnp.bfloat16).reshape(nb, DIM)
        outs.append(_score_call(u, t, w1, w2, bp, nb))
    return jnp.concatenate(outs) if NSLICES > 1 else outs[0]


# speedup vs baseline: 2.2337x; 2.2337x over previous
import functools

import jax
import jax.numpy as jnp
from jax import lax
from jax.experimental import pallas as pl
from jax.experimental.pallas import tpu as pltpu
from jax.experimental.pallas import tpu_sc as plsc

B = 16384
DIM = 128


def _sc_info():
    try:
        info = plsc.get_sparse_core_info()
        return info.num_cores, info.num_subcores
    except Exception:
        return 2, 16


def _make_gather(nb):
    NC, NS = _sc_info()
    NW = NC * NS
    b_per_w = nb // NW
    C = min(64, b_per_w)
    n_chunks = b_per_w // C

    mesh = plsc.VectorSubcoreMesh(core_axis_name="c", subcore_axis_name="s")

    @functools.partial(
        pl.kernel,
        mesh=mesh,
        compiler_params=pltpu.CompilerParams(needs_layout_passes=False),
        out_type=[
            jax.ShapeDtypeStruct((nb // 2, DIM), jnp.int32),
            jax.ShapeDtypeStruct((nb // 2, DIM), jnp.int32),
        ],
        scratch_types=[
            pltpu.VMEM((n_chunks, C), jnp.int32),
            pltpu.VMEM((n_chunks, C), jnp.int32),
            pltpu.VMEM((n_chunks, C), jnp.int32),
            pltpu.VMEM((C, DIM), jnp.float32),
            pltpu.VMEM((C, DIM), jnp.float32),
            pltpu.VMEM((C, DIM), jnp.float32),
            pltpu.VMEM((C // 2, DIM), jnp.int32),
            pltpu.VMEM((C // 2, DIM), jnp.int32),
            pltpu.VMEM((C, DIM), jnp.float32),
            pltpu.VMEM((C, DIM), jnp.float32),
            pltpu.VMEM((C, DIM), jnp.float32),
            pltpu.VMEM((C // 2, DIM), jnp.int32),
            pltpu.VMEM((C // 2, DIM), jnp.int32),
            pltpu.SemaphoreType.DMA,
            pltpu.SemaphoreType.DMA,
            pltpu.SemaphoreType.DMA,
            pltpu.SemaphoreType.DMA,
            pltpu.SemaphoreType.DMA,
        ],
    )
    def gather_kernel(heads_hbm, rels_hbm, tails_hbm, ent_hbm, rel_hbm,
                      u_out, t_out, hidx, ridx, tidx,
                      hv0, rv0, tv0, ub0, tb0,
                      hv1, rv1, tv1, ub1, tb1,
                      isem, gsem0, gsem1, wsem0, wsem1):
        wid = lax.axis_index("s") * NC + lax.axis_index("c")
        base = wid * b_per_w
        slots = [(hv0, rv0, tv0, ub0, tb0, gsem0, wsem0),
                 (hv1, rv1, tv1, ub1, tb1, gsem1, wsem1)]

        idescs = []
        for ci in range(n_chunks):
            off = base + ci * C
            idescs.append(pltpu.async_copy(
                heads_hbm.at[pl.ds(off, C)], hidx.at[ci], isem))
            idescs.append(pltpu.async_copy(
                rels_hbm.at[pl.ds(off, C)], ridx.at[ci], isem))
            idescs.append(pltpu.async_copy(
                tails_hbm.at[pl.ds(off, C)], tidx.at[ci], isem))
        for d in idescs:
            d.wait()

        def fire(ci):
            hv, rv, tv, _, _, gsem, _ = slots[ci % 2]
            return (pltpu.async_copy(ent_hbm.at[hidx.at[ci]], hv, gsem),
                    pltpu.async_copy(rel_hbm.at[ridx.at[ci]], rv, gsem),
                    pltpu.async_copy(ent_hbm.at[tidx.at[ci]], tv, gsem))

        def lo16(x):
            v = plsc.bitcast(x, jnp.uint32)
            return (v + jnp.uint32(0x8000)) >> jnp.uint32(16)

        def hi16(x):
            v = plsc.bitcast(x, jnp.uint32)
            return (v + jnp.uint32(0x8000)) & jnp.uint32(0xFFFF0000)

        gd = {0: fire(0)}
        wd = {}
        for ci in range(n_chunks):
            hv, rv, tv, ub, tb, _, wsem = slots[ci % 2]
            off = base + ci * C
            if ci - 1 in wd:
                for d in wd.pop(ci - 1):
                    d.wait()
            if ci + 1 < n_chunks:
                gd[ci + 1] = fire(ci + 1)
            for d in gd.pop(ci):
                d.wait()

            def pair_body(i, carry):
                i0 = 2 * i
                i1 = 2 * i + 1
                for j in range(DIM // 16):
                    sj = pl.ds(16 * j, 16)
                    ua = hv[i0, sj] * rv[i0, sj]
                    uo = hv[i1, sj] * rv[i1, sj]
                    uw = lo16(ua) | hi16(uo)
                    ub[i, sj] = plsc.bitcast(uw, jnp.int32)
                    tw = lo16(tv[i0, sj]) | hi16(tv[i1, sj])
                    tb[i, sj] = plsc.bitcast(tw, jnp.int32)
                return carry
            lax.fori_loop(0, C // 2, pair_body, 0, unroll=2)

            off2 = wid * (b_per_w // 2) + ci * (C // 2)
            wu = pltpu.async_copy(ub, u_out.at[pl.ds(off2, C // 2)], wsem)
            wt = pltpu.async_copy(tb, t_out.at[pl.ds(off2, C // 2)], wsem)
            wd[ci] = (wu, wt)
        for ci in list(wd):
            for d in wd.pop(ci):
                d.wait()

    return gather_kernel


def _score_body(u_ref, t_ref, w1_ref, w2_ref, b_ref, ones_ref, out_ref):
    u = pltpu.bitcast(u_ref[...], jnp.bfloat16)
    t = pltpu.bitcast(t_ref[...], jnp.bfloat16)
    acc = jnp.dot(u, w1_ref[...], preferred_element_type=jnp.float32)
    acc = acc + jnp.dot(t, w2_ref[...], preferred_element_type=jnp.float32)
    acc = acc + b_ref[...]
    m = acc * t.astype(jnp.float32)
    out_ref[...] = jnp.dot(m, ones_ref[...],
                           preferred_element_type=jnp.float32)


def _score_call(u32, t32, w1, w2, b, nb):
    RB = min(4096, nb)
    ones = jnp.ones((DIM, 8), jnp.float32)
    out2d = pl.pallas_call(
        _score_body,
        grid=(nb // RB,),
        in_specs=[
            pl.BlockSpec((RB // 2, DIM), lambda i: (i, 0)),
            pl.BlockSpec((RB // 2, DIM), lambda i: (i, 0)),
            pl.BlockSpec((DIM, DIM), lambda i: (0, 0)),
            pl.BlockSpec((DIM, DIM), lambda i: (0, 0)),
            pl.BlockSpec((DIM,), lambda i: (0,)),
            pl.BlockSpec((DIM, 8), lambda i: (0, 0)),
        ],
        out_specs=pl.BlockSpec((RB, 8), lambda i: (i, 0)),
        out_shape=jax.ShapeDtypeStruct((nb, 8), jnp.float32),
    )(u32, t32, w1, w2, b, ones)
    return out2d[:, 0]


NSLICES = 1


@jax.jit
def kernel(heads, rels, tails, entity_emb, rel_emb, W, b):
    nb = B // NSLICES
    gather = _make_gather(nb)
    w1 = W[:DIM].astype(jnp.bfloat16)
    w2 = W[DIM:].astype(jnp.bfloat16)
    outs = []
    for k in range(NSLICES):
        sl = slice(k * nb, (k + 1) * nb)
        u32, t32 = gather(heads[sl], rels[sl], tails[sl],
                          entity_emb, rel_emb)
        outs.append(_score_call(u32, t32, w1, w2, b, nb))
    return jnp.concatenate(outs) if NSLICES > 1 else outs[0]


# speedup vs baseline: 2.5445x; 1.1391x over previous
import functools

import jax
import jax.numpy as jnp
from jax import lax
from jax.experimental import pallas as pl
from jax.experimental.pallas import tpu as pltpu
from jax.experimental.pallas import tpu_sc as plsc

B = 16384
DIM = 128


def _sc_info():
    try:
        info = plsc.get_sparse_core_info()
        return info.num_cores, info.num_subcores
    except Exception:
        return 2, 16


def _make_gather(nb):
    NC, NS = _sc_info()
    NW = NC * NS
    b_per_w = nb // NW
    C = min(128, b_per_w)
    n_chunks = b_per_w // C

    mesh = plsc.VectorSubcoreMesh(core_axis_name="c", subcore_axis_name="s")

    @functools.partial(
        pl.kernel,
        mesh=mesh,
        out_type=[
            jax.ShapeDtypeStruct((nb, DIM), jnp.float32),
            jax.ShapeDtypeStruct((nb, DIM), jnp.float32),
        ],
        scratch_types=[
            pltpu.VMEM((n_chunks, C), jnp.int32),
            pltpu.VMEM((n_chunks, C), jnp.int32),
            pltpu.VMEM((n_chunks, C), jnp.int32),
            pltpu.VMEM((C, DIM), jnp.float32),
            pltpu.VMEM((C, DIM), jnp.float32),
            pltpu.VMEM((C, DIM), jnp.float32),
            pltpu.VMEM((C, DIM), jnp.float32),
            pltpu.VMEM((C, DIM), jnp.float32),
            pltpu.VMEM((C, DIM), jnp.float32),
            pltpu.SemaphoreType.DMA,
            pltpu.SemaphoreType.DMA,
            pltpu.SemaphoreType.DMA,
            pltpu.SemaphoreType.DMA,
            pltpu.SemaphoreType.DMA,
        ],
    )
    def gather_kernel(heads_hbm, rels_hbm, tails_hbm, ent_hbm, rel_hbm,
                      u_out, t_out, hidx, ridx, tidx,
                      hv0, rv0, tv0, hv1, rv1, tv1,
                      isem, gsem0, gsem1, wsem0, wsem1):
        wid = lax.axis_index("s") * NC + lax.axis_index("c")
        base = wid * b_per_w
        slots = [(hv0, rv0, tv0, gsem0, wsem0),
                 (hv1, rv1, tv1, gsem1, wsem1)]

        idescs = []
        for ci in range(n_chunks):
            off = base + ci * C
            idescs.append(pltpu.async_copy(
                heads_hbm.at[pl.ds(off, C)], hidx.at[ci], isem))
            idescs.append(pltpu.async_copy(
                rels_hbm.at[pl.ds(off, C)], ridx.at[ci], isem))
            idescs.append(pltpu.async_copy(
                tails_hbm.at[pl.ds(off, C)], tidx.at[ci], isem))
        for d in idescs:
            d.wait()

        def fire(ci):
            hv, rv, tv, gsem, _ = slots[ci % 2]
            return (pltpu.async_copy(ent_hbm.at[hidx.at[ci]], hv, gsem),
                    pltpu.async_copy(rel_hbm.at[ridx.at[ci]], rv, gsem),
                    pltpu.async_copy(ent_hbm.at[tidx.at[ci]], tv, gsem))

        gd = {0: fire(0)}
        wd = {}
        for ci in range(n_chunks):
            hv, rv, tv, _, wsem = slots[ci % 2]
            off = base + ci * C
            if ci - 1 in wd:
                for d in wd.pop(ci - 1):
                    d.wait()
            if ci + 1 < n_chunks:
                gd[ci + 1] = fire(ci + 1)
            for d in gd.pop(ci):
                d.wait()
            wt = pltpu.async_copy(tv, t_out.at[pl.ds(off, C)], wsem)

            def mul_body(i, carry):
                for j in range(DIM // 16):
                    sl = pl.ds(j * 16, 16)
                    hv[i, sl] = hv[i, sl] * rv[i, sl]
                return carry
            lax.fori_loop(0, C, mul_body, 0, unroll=2)

            wu = pltpu.async_copy(hv, u_out.at[pl.ds(off, C)], wsem)
            wd[ci] = (wt, wu)
        for ci in list(wd):
            for d in wd.pop(ci):
                d.wait()

    return gather_kernel


def _score_body(u_ref, t_ref, w1_ref, w2_ref, b_ref, ones_ref, out_ref):
    u = u_ref[...]
    t = t_ref[...]
    acc = jnp.dot(u, w1_ref[...], preferred_element_type=jnp.float32)
    acc = acc + jnp.dot(t, w2_ref[...], preferred_element_type=jnp.float32)
    acc = acc + b_ref[...]
    out_ref[...] = jnp.dot(acc * t, ones_ref[...],
                           preferred_element_type=jnp.float32)


def _score_call(u, t, w1, w2, b, nb):
    RB = min(4096, nb)
    ones = jnp.ones((DIM, 8), jnp.float32)
    out2d = pl.pallas_call(
        _score_body,
        grid=(nb // RB,),
        in_specs=[
            pl.BlockSpec((RB, DIM), lambda i: (i, 0)),
            pl.BlockSpec((RB, DIM), lambda i: (i, 0)),
            pl.BlockSpec((DIM, DIM), lambda i: (0, 0)),
            pl.BlockSpec((DIM, DIM), lambda i: (0, 0)),
            pl.BlockSpec((DIM,), lambda i: (0,)),
            pl.BlockSpec((DIM, 8), lambda i: (0, 0)),
        ],
        out_specs=pl.BlockSpec((RB, 8), lambda i: (i, 0)),
        out_shape=jax.ShapeDtypeStruct((nb, 8), jnp.float32),
    )(u, t, w1, w2, b, ones)
    return out2d[:, 0]


NSLICES = 1


@jax.jit
def kernel(heads, rels, tails, entity_emb, rel_emb, W, b):
    nb = B // NSLICES
    gather = _make_gather(nb)
    w1 = W[:DIM]
    w2 = W[DIM:]
    outs = []
    for k in range(NSLICES):
        sl = slice(k * nb, (k + 1) * nb)
        u, t = gather(heads[sl], rels[sl], tails[sl], entity_emb, rel_emb)
        outs.append(_score_call(u, t, w1, w2, b, nb))
    return jnp.concatenate(outs) if NSLICES > 1 else outs[0]


# speedup vs baseline: 2.5640x; 1.0077x over previous
import functools

import jax
import jax.numpy as jnp
from jax import lax
from jax.experimental import pallas as pl
from jax.experimental.pallas import tpu as pltpu
from jax.experimental.pallas import tpu_sc as plsc

B = 16384
DIM = 128


def _sc_info():
    try:
        info = plsc.get_sparse_core_info()
        return info.num_cores, info.num_subcores
    except Exception:
        return 2, 16


def _make_gather(nb):
    NC, NS = _sc_info()
    NW = NC * NS
    b_per_w = nb // NW
    C = min(64, b_per_w)
    n_chunks = b_per_w // C
    S = 3 if n_chunks >= 3 else 2

    mesh = plsc.VectorSubcoreMesh(core_axis_name="c", subcore_axis_name="s")

    @functools.partial(
        pl.kernel,
        mesh=mesh,
        out_type=[
            jax.ShapeDtypeStruct((nb, DIM), jnp.float32),
            jax.ShapeDtypeStruct((nb, DIM), jnp.float32),
        ],
        scratch_types=[
            pltpu.VMEM((n_chunks, C), jnp.int32),
            pltpu.VMEM((n_chunks, C), jnp.int32),
            pltpu.VMEM((n_chunks, C), jnp.int32),
        ] + [pltpu.VMEM((C, DIM), jnp.float32)
             for _ in range(3 * S)]
          + [pltpu.SemaphoreType.DMA]
          + [pltpu.SemaphoreType.DMA
             for _ in range(S)]
          + [pltpu.SemaphoreType.DMA
             for _ in range(S)],
    )
    def gather_kernel(heads_hbm, rels_hbm, tails_hbm, ent_hbm, rel_hbm,
                      u_out, t_out, hidx, ridx, tidx, *rest):
        bufs, rest = rest[:3 * S], rest[3 * S:]
        isem = rest[0]
        gsems = rest[1:1 + S]
        wsems = rest[1 + S:1 + 2 * S]
        wid = lax.axis_index("s") * NC + lax.axis_index("c")
        base = wid * b_per_w
        slots = [(bufs[3 * s], bufs[3 * s + 1], bufs[3 * s + 2],
                  gsems[s], wsems[s]) for s in range(S)]

        idescs = []
        for ci in range(n_chunks):
            off = base + ci * C
            idescs.append(pltpu.async_copy(
                heads_hbm.at[pl.ds(off, C)], hidx.at[ci], isem))
            idescs.append(pltpu.async_copy(
                rels_hbm.at[pl.ds(off, C)], ridx.at[ci], isem))
            idescs.append(pltpu.async_copy(
                tails_hbm.at[pl.ds(off, C)], tidx.at[ci], isem))
        for d in idescs:
            d.wait()

        def fire(ci):
            hv, rv, tv, gsem, _ = slots[ci % S]
            return (pltpu.async_copy(ent_hbm.at[hidx.at[ci]], hv, gsem),
                    pltpu.async_copy(rel_hbm.at[ridx.at[ci]], rv, gsem),
                    pltpu.async_copy(ent_hbm.at[tidx.at[ci]], tv, gsem))

        gd = {0: fire(0)}
        if n_chunks > 1:
            gd[1] = fire(1)
        wd = {}
        for ci in range(n_chunks):
            hv, rv, tv, _, wsem = slots[ci % S]
            off = base + ci * C
            if ci + 2 < n_chunks:
                prev = ci + 2 - S
                if prev in wd:
                    for d in wd.pop(prev):
                        d.wait()
                gd[ci + 2] = fire(ci + 2)
            for d in gd.pop(ci):
                d.wait()
            wt = pltpu.async_copy(tv, t_out.at[pl.ds(off, C)], wsem)

            def mul_body(i, carry):
                for j in range(DIM // 16):
                    sl = pl.ds(j * 16, 16)
                    hv[i, sl] = hv[i, sl] * rv[i, sl]
                return carry
            lax.fori_loop(0, C, mul_body, 0, unroll=2)

            wu = pltpu.async_copy(hv, u_out.at[pl.ds(off, C)], wsem)
            wd[ci] = (wt, wu)
        for ci in list(wd):
            for d in wd.pop(ci):
                d.wait()

    return gather_kernel


def _score_body(u_ref, t_ref, w1_ref, w2_ref, b_ref, ones_ref, out_ref):
    u = u_ref[...]
    t = t_ref[...]
    acc = jnp.dot(u, w1_ref[...], preferred_element_type=jnp.float32)
    acc = acc + jnp.dot(t, w2_ref[...], preferred_element_type=jnp.float32)
    acc = acc + b_ref[...]
    out_ref[...] = jnp.dot(acc * t, ones_ref[...],
                           preferred_element_type=jnp.float32)


def _score_call(u, t, w1, w2, b, nb):
    RB = min(4096, nb)
    ones = jnp.ones((DIM, 8), jnp.float32)
    out2d = pl.pallas_call(
        _score_body,
        grid=(nb // RB,),
        in_specs=[
            pl.BlockSpec((RB, DIM), lambda i: (i, 0)),
            pl.BlockSpec((RB, DIM), lambda i: (i, 0)),
            pl.BlockSpec((DIM, DIM), lambda i: (0, 0)),
            pl.BlockSpec((DIM, DIM), lambda i: (0, 0)),
            pl.BlockSpec((DIM,), lambda i: (0,)),
            pl.BlockSpec((DIM, 8), lambda i: (0, 0)),
        ],
        out_specs=pl.BlockSpec((RB, 8), lambda i: (i, 0)),
        out_shape=jax.ShapeDtypeStruct((nb, 8), jnp.float32),
    )(u, t, w1, w2, b, ones)
    return out2d[:, 0]


NSLICES = 1


@jax.jit
def kernel(heads, rels, tails, entity_emb, rel_emb, W, b):
    nb = B // NSLICES
    gather = _make_gather(nb)
    w1 = W[:DIM]
    w2 = W[DIM:]
    outs = []
    for k in range(NSLICES):
        sl = slice(k * nb, (k + 1) * nb)
        u, t = gather(heads[sl], rels[sl], tails[sl], entity_emb, rel_emb)
        outs.append(_score_call(u, t, w1, w2, b, nb))
    return jnp.concatenate(outs) if NSLICES > 1 else outs[0]
